# baseline (speedup 1.0000x reference)
import functools
import math

import jax
import jax.numpy as jnp
from jax import lax
from jax.experimental import pallas as pl
from jax.experimental.pallas import tpu as pltpu
from jax.experimental.pallas import tpu_sc as plsc

D_MODEL = 64
SCALE = math.sqrt(D_MODEL)
LANES = 16
NUM_CORES = 2
NUM_SUBCORES = 16
NUM_WORKERS = NUM_CORES * NUM_SUBCORES
GROUP = 128
TC_PER_TASK = 2
TASK_ROWS = GROUP * TC_PER_TASK
SUBL = 8


def kernel(x, table):
    b0, s = x.shape
    batch = b0 * s
    n_tc = b0 // GROUP
    tasks_per_seq = n_tc // TC_PER_TASK
    n_tasks = s * tasks_per_seq
    tasks_per_worker = n_tasks // NUM_WORKERS
    n_tr = D_MODEL // SUBL

    x_t = x.T.astype(jnp.int32)

    mesh = plsc.VectorSubcoreMesh(core_axis_name="c", subcore_axis_name="s")

    @functools.partial(
        pl.kernel,
        out_type=jax.ShapeDtypeStruct((s, n_tr, n_tc, SUBL, GROUP),
                                      jnp.float32),
        mesh=mesh,
        scratch_types=[
            pltpu.VMEM((TASK_ROWS,), jnp.int32),
            pltpu.VMEM((TASK_ROWS,), jnp.int32),
            pltpu.VMEM((TASK_ROWS, D_MODEL), jnp.float32),
            pltpu.VMEM((TASK_ROWS, D_MODEL), jnp.float32),
            pltpu.VMEM((n_tr, TC_PER_TASK, SUBL, GROUP), jnp.float32),
            pltpu.VMEM((n_tr, TC_PER_TASK, SUBL, GROUP), jnp.float32),
            pltpu.SemaphoreType.DMA,
            pltpu.SemaphoreType.DMA,
            pltpu.SemaphoreType.DMA,
            pltpu.SemaphoreType.DMA,
            pltpu.SemaphoreType.DMA,
            pltpu.SemaphoreType.DMA,
        ],
        compiler_params=pltpu.CompilerParams(
            use_tc_tiling_on_sc=False, needs_layout_passes=False),
    )
    def emb_kernel(x_hbm, table_hbm, out_hbm,
                   idx_v0, idx_v1, rows_v0, rows_v1, out_v0, out_v1,
                   i_sem0, i_sem1, g_sem0, g_sem1, o_sem0, o_sem1):
        idx_v = (idx_v0, idx_v1)
        rows_v = (rows_v0, rows_v1)
        out_v = (out_v0, out_v1)
        i_sem = (i_sem0, i_sem1)
        g_sem = (g_sem0, g_sem1)
        o_sem = (o_sem0, o_sem1)

        wid = lax.axis_index("s") * NUM_CORES + lax.axis_index("c")
        base_task = wid * tasks_per_worker

        iota = lax.iota(jnp.int32, LANES)

        def start_idx(t, slot):
            sq = t // tasks_per_seq
            tp = t % tasks_per_seq
            pltpu.async_copy(
                x_hbm.at[sq, pl.ds(tp * TASK_ROWS, TASK_ROWS)],
                idx_v[slot], i_sem[slot],
            )

        def wait_idx(slot):
            pltpu.make_async_copy(
                x_hbm.at[0, pl.ds(0, TASK_ROWS)], idx_v[slot], i_sem[slot]
            ).wait()

        def fire_gathers(slot):
            for j in range(TC_PER_TASK):
                pltpu.async_copy(
                    table_hbm.at[idx_v[slot].at[pl.ds(j * GROUP, GROUP)]],
                    rows_v[slot].at[pl.ds(j * GROUP, GROUP)],
                    g_sem[slot],
                )

        def wait_gathers(slot):
            pltpu.make_async_copy(
                table_hbm.at[pl.ds(0, TASK_ROWS)], rows_v[slot], g_sem[slot]
            ).wait()

        def transpose_scale(slot):
            rows = rows_v[slot]
            outb = out_v[slot]
            for tcl in range(TC_PER_TASK):
                for g8 in range(SUBL):
                    j_vec = iota + (tcl * GROUP + g8 * LANES)

                    @plsc.parallel_loop(0, D_MODEL, unroll=8)
                    def _(k):
                        col = jnp.full((LANES,), 0, jnp.int32) + k
                        v = plsc.load_gather(rows, [j_vec, col])
                        tr = k // SUBL
                        k8 = k % SUBL
                        outb[tr, tcl, k8, pl.ds(g8 * LANES, LANES)] = (
                            v * SCALE
                        )

        def fire_out(t, slot):
            sq = t // tasks_per_seq
            tp = t % tasks_per_seq
            for tr in range(n_tr):
                pltpu.async_copy(
                    out_v[slot].at[tr],
                    out_hbm.at[sq, tr, pl.ds(tp * TC_PER_TASK, TC_PER_TASK)],
                    o_sem[slot],
                )

        def wait_out(slot):
            pltpu.make_async_copy(
                out_v[slot], out_hbm.at[0, pl.ds(0, n_tr), pl.ds(0, TC_PER_TASK)],
                o_sem[slot],
            ).wait()

        pltpu.sync_copy(
            x_hbm.at[base_task // tasks_per_seq,
                     pl.ds((base_task % tasks_per_seq) * TASK_ROWS, TASK_ROWS)],
            idx_v[0],
        )
        fire_gathers(0)
        start_idx(base_task + 1, 1)
        wait_idx(1)
        fire_gathers(1)
        wait_gathers(0)
        start_idx(base_task + 2, 0)
        transpose_scale(0)
        fire_out(base_task, 0)

        def steady(t, slot):
            other = 1 - slot
            wait_out(other)
            wait_idx(other)
            fire_gathers(other)
            wait_gathers(slot)

            @pl.when(t + 2 < base_task + tasks_per_worker)
            def _():
                start_idx(t + 2, slot)

            transpose_scale(slot)
            fire_out(t, slot)

        @pl.loop(0, (tasks_per_worker - 2) // 2)
        def _(i):
            t = base_task + 1 + i * 2
            steady(t, 1)
            steady(t + 1, 0)

        t_last = base_task + tasks_per_worker - 1
        wait_out(0)
        wait_gathers(1)
        transpose_scale(1)
        fire_out(t_last, 1)
        wait_out(1)

    out5d = emb_kernel(x_t, table)
    out = out5d.transpose(2, 4, 0, 1, 3).reshape(b0, s, D_MODEL)
    return out


# speedup vs baseline: 1.7467x; 1.7467x over previous
import functools
import math

import jax
import jax.numpy as jnp
from jax import lax
from jax.experimental import pallas as pl
from jax.experimental.pallas import tpu as pltpu
from jax.experimental.pallas import tpu_sc as plsc

D_MODEL = 64
SCALE = math.sqrt(D_MODEL)
LANES = 16
NUM_CORES = 2
NUM_SUBCORES = 16
NUM_WORKERS = NUM_CORES * NUM_SUBCORES
GROUP = 128
TC_PER_TASK = 2
TASK_ROWS = GROUP * TC_PER_TASK
SUBL = 8
PAD_W = TASK_ROWS + 1


def kernel(x, table):
    b0, s = x.shape
    batch = b0 * s
    n_tc = b0 // GROUP
    tasks_per_seq = n_tc // TC_PER_TASK
    n_tasks = s * tasks_per_seq
    tasks_per_worker = n_tasks // NUM_WORKERS
    n_tr = D_MODEL // SUBL

    x_t = x.T.astype(jnp.int32)

    mesh = plsc.VectorSubcoreMesh(core_axis_name="c", subcore_axis_name="s")

    @functools.partial(
        pl.kernel,
        out_type=jax.ShapeDtypeStruct((s, n_tr, n_tc, SUBL, GROUP),
                                      jnp.float32),
        mesh=mesh,
        scratch_types=[
            pltpu.VMEM((TASK_ROWS,), jnp.int32),
            pltpu.VMEM((TASK_ROWS,), jnp.int32),
            pltpu.VMEM((TASK_ROWS, D_MODEL), jnp.float32),
            pltpu.VMEM((TASK_ROWS, D_MODEL), jnp.float32),
            pltpu.VMEM((D_MODEL, PAD_W), jnp.float32),
            pltpu.VMEM((D_MODEL, PAD_W), jnp.float32),
            pltpu.SemaphoreType.DMA,
            pltpu.SemaphoreType.DMA,
            pltpu.SemaphoreType.DMA,
            pltpu.SemaphoreType.DMA,
            pltpu.SemaphoreType.DMA,
            pltpu.SemaphoreType.DMA,
        ],
        compiler_params=pltpu.CompilerParams(
            use_tc_tiling_on_sc=False, needs_layout_passes=False),
    )
    def emb_kernel(x_hbm, table_hbm, out_hbm,
                   idx_v0, idx_v1, rows_v0, rows_v1, out_v0, out_v1,
                   i_sem0, i_sem1, g_sem0, g_sem1, o_sem0, o_sem1):
        idx_v = (idx_v0, idx_v1)
        rows_v = (rows_v0, rows_v1)
        out_v = (out_v0, out_v1)
        i_sem = (i_sem0, i_sem1)
        g_sem = (g_sem0, g_sem1)
        o_sem = (o_sem0, o_sem1)

        wid = lax.axis_index("s") * NUM_CORES + lax.axis_index("c")
        base_task = wid * tasks_per_worker

        iota = lax.iota(jnp.int32, LANES)
        row_vecs = [iota + (m * LANES) for m in range(D_MODEL // LANES)]

        def start_idx(t, slot):
            sq = t // tasks_per_seq
            tp = t % tasks_per_seq
            pltpu.async_copy(
                x_hbm.at[sq, pl.ds(tp * TASK_ROWS, TASK_ROWS)],
                idx_v[slot], i_sem[slot],
            )

        def wait_idx(slot):
            pltpu.make_async_copy(
                x_hbm.at[0, pl.ds(0, TASK_ROWS)], idx_v[slot], i_sem[slot]
            ).wait()

        def fire_gathers(slot):
            for j in range(TC_PER_TASK):
                pltpu.async_copy(
                    table_hbm.at[idx_v[slot].at[pl.ds(j * GROUP, GROUP)]],
                    rows_v[slot].at[pl.ds(j * GROUP, GROUP)],
                    g_sem[slot],
                )

        def wait_gathers(slot):
            pltpu.make_async_copy(
                table_hbm.at[pl.ds(0, TASK_ROWS)], rows_v[slot], g_sem[slot]
            ).wait()

        def transpose_scale(slot):
            rows = rows_v[slot]
            outb = out_v[slot]

            @plsc.parallel_loop(0, TASK_ROWS, unroll=4)
            def _(j):
                col = jnp.full((LANES,), 0, jnp.int32) + j
                for m in range(D_MODEL // LANES):
                    v = rows[j, pl.ds(m * LANES, LANES)]
                    plsc.store_scatter(outb, [row_vecs[m], col], v * SCALE)

        def fire_out(t, slot):
            sq = t // tasks_per_seq
            tp = t % tasks_per_seq
            for tr in range(n_tr):
                for tcl in range(TC_PER_TASK):
                    pltpu.async_copy(
                        out_v[slot].at[pl.ds(tr * SUBL, SUBL),
                                       pl.ds(tcl * GROUP, GROUP)],
                        out_hbm.at[sq, tr, tp * TC_PER_TASK + tcl],
                        o_sem[slot],
                    )

        def wait_out(slot):
            for tr in range(n_tr):
                for tcl in range(TC_PER_TASK):
                    pltpu.make_async_copy(
                        out_v[slot].at[pl.ds(tr * SUBL, SUBL),
                                       pl.ds(tcl * GROUP, GROUP)],
                        out_hbm.at[0, tr, tcl],
                        o_sem[slot],
                    ).wait()

        pltpu.sync_copy(
            x_hbm.at[base_task // tasks_per_seq,
                     pl.ds((base_task % tasks_per_seq) * TASK_ROWS, TASK_ROWS)],
            idx_v[0],
        )
        fire_gathers(0)
        start_idx(base_task + 1, 1)
        wait_idx(1)
        fire_gathers(1)
        wait_gathers(0)
        start_idx(base_task + 2, 0)
        transpose_scale(0)
        fire_out(base_task, 0)

        def steady(t, slot):
            other = 1 - slot
            wait_out(other)
            wait_idx(other)
            fire_gathers(other)
            wait_gathers(slot)

            @pl.when(t + 2 < base_task + tasks_per_worker)
            def _():
                start_idx(t + 2, slot)

            transpose_scale(slot)
            fire_out(t, slot)

        @pl.loop(0, (tasks_per_worker - 2) // 2)
        def _(i):
            t = base_task + 1 + i * 2
            steady(t, 1)
            steady(t + 1, 0)

        t_last = base_task + tasks_per_worker - 1
        wait_out(0)
        wait_gathers(1)
        transpose_scale(1)
        fire_out(t_last, 1)
        wait_out(1)

    out5d = emb_kernel(x_t, table)
    out = out5d.transpose(2, 4, 0, 1, 3).reshape(b0, s, D_MODEL)
    return out
